# baseline (speedup 1.0000x reference)
import jax
import jax.numpy as jnp
from jax.experimental import pallas as pl
from jax.experimental.pallas import tpu as pltpu

MAX_NUM_TILES = 4
HIDDEN_SIZE = 1280
NUM_PATCHES = 1025
DEPTH = 4


def _body(ids_ref, gate_ref, table_ref, hid_ref, out_ref,
          in_buf, out_buf, in_sem, out_sem):
    n = hid_ref.shape[0]
    g = jnp.tanh(gate_ref[0])

    def in_copy(i):
        return pltpu.make_async_copy(
            hid_ref.at[i], in_buf.at[i % DEPTH], in_sem.at[i % DEPTH])

    def out_copy(i):
        return pltpu.make_async_copy(
            out_buf.at[i % DEPTH], out_ref.at[i], out_sem.at[i % DEPTH])

    for i in range(min(DEPTH, n)):
        in_copy(i).start()

    for i in range(n):
        s = i % DEPTH
        in_copy(i).wait()
        if i >= DEPTH:
            out_copy(i - DEPTH).wait()
        row = ids_ref[i // MAX_NUM_TILES]
        t = i % MAX_NUM_TILES
        emb = table_ref[row, t * HIDDEN_SIZE:(t + 1) * HIDDEN_SIZE]
        out_buf[s] = in_buf[s] + (g * emb)[None, :]
        out_copy(i).start()
        if i + DEPTH < n:
            in_copy(i + DEPTH).start()

    for i in range(max(0, n - DEPTH), n):
        out_copy(i).wait()


def kernel(hidden_state, aspect_ratio_ids, embedding_table, gate):
    batch = hidden_state.shape[0]
    n = batch * MAX_NUM_TILES
    ids = aspect_ratio_ids.astype(jnp.int32)
    hid3d = hidden_state.reshape(n, NUM_PATCHES, HIDDEN_SIZE)

    out = pl.pallas_call(
        _body,
        in_specs=[
            pl.BlockSpec(memory_space=pltpu.SMEM),
            pl.BlockSpec(memory_space=pltpu.SMEM),
            pl.BlockSpec(memory_space=pltpu.VMEM),
            pl.BlockSpec(memory_space=pltpu.MemorySpace.HBM),
        ],
        out_specs=pl.BlockSpec(memory_space=pltpu.MemorySpace.HBM),
        out_shape=jax.ShapeDtypeStruct(hid3d.shape, hid3d.dtype),
        scratch_shapes=[
            pltpu.VMEM((DEPTH, NUM_PATCHES, HIDDEN_SIZE), jnp.float32),
            pltpu.VMEM((DEPTH, NUM_PATCHES, HIDDEN_SIZE), jnp.float32),
            pltpu.SemaphoreType.DMA((DEPTH,)),
            pltpu.SemaphoreType.DMA((DEPTH,)),
        ],
    )(ids, gate, embedding_table, hid3d)
    return out.reshape(hidden_state.shape)
